# baseline (speedup 1.0000x reference)
import functools

import jax
import jax.numpy as jnp
from jax import lax
from jax.experimental import pallas as pl
from jax.experimental.pallas import tpu as pltpu
from jax.experimental.pallas import tpu_sc as plsc

N = 325
NROWPAD = 328
NPAD = 336
TOPK = 16
LANES = 16
NCH = NPAD // LANES
NW = 32


def _tc_graph_body(x_ref, w_ref, o_ref):
    x = x_ref[0]
    a1 = jnp.dot(x, w_ref[0], preferred_element_type=jnp.float32)
    a2 = jnp.dot(x, w_ref[1], preferred_element_type=jnp.float32)
    g = lax.dot_general(a1, a2, (((1,), (1,)), ((), ())),
                        preferred_element_type=jnp.float32)
    g = jnp.maximum(g, 0.0)
    o_ref[...] = jnp.pad(g, ((0, NROWPAD - N), (0, NPAD - N)))


def _make_sc(nslices, batch):
    assert batch % 8 == 0
    r_total = nslices * NROWPAD
    nbatch = r_total // batch
    assert nbatch * batch == r_total
    nb_per_w = -(-nbatch // NW)

    def _sc_body(g_hbm, out_hbm, in_v, out_v):
        wid = lax.axis_index("s") * 2 + lax.axis_index("c")

        def batch_body(bi, carry):
            k = bi * NW + wid

            @pl.when(k < nbatch)
            def _():
                _do_batch(k)
            return carry

        def _do_batch(k):
            rb = k * batch
            pltpu.sync_copy(g_hbm.at[pl.ds(rb, batch)], in_v)

            def row_body(r, c2):
                chunks = [in_v[r, pl.ds(c * LANES, LANES)]
                          for c in range(NCH)]
                m = chunks[0]
                for c in range(1, NCH):
                    m = jnp.maximum(m, chunks[c])
                mx = jnp.max(m)
                top, _ = plsc.sort_key_val(chunks[0], chunks[0])
                for c in range(1, NCH):
                    v = chunks[c]
                    dsc, _ = plsc.sort_key_val(v, v, descending=True)
                    cand = jnp.maximum(top, dsc)
                    top, _ = plsc.sort_key_val(cand, cand)
                t = jnp.min(top)
                acc = jnp.zeros((LANES,), jnp.float32)
                es = []
                for c in range(NCH):
                    v = chunks[c]
                    keep = jnp.logical_and(v >= t, v > 0.0)
                    e = jnp.where(keep, jnp.exp(v - mx), 0.0)
                    acc = acc + e
                    es.append(e)
                den = jnp.broadcast_to(jnp.sum(acc) + 1e-5, (LANES,))
                inv = jnp.ones((LANES,), jnp.float32) / den
                for c in range(NCH):
                    out_v[r, pl.ds(c * LANES, LANES)] = es[c] * inv
                return c2

            lax.fori_loop(0, batch, row_body, 0)
            pltpu.sync_copy(out_v, out_hbm.at[pl.ds(rb, batch)])

        lax.fori_loop(0, nb_per_w, batch_body, 0)

    return pl.kernel(
        _sc_body,
        out_type=jax.ShapeDtypeStruct((r_total, NPAD), jnp.float32),
        mesh=plsc.VectorSubcoreMesh(core_axis_name="c",
                                    subcore_axis_name="s"),
        scratch_types=[
            pltpu.VMEM((batch, NPAD), jnp.float32),
            pltpu.VMEM((batch, NPAD), jnp.float32),
        ],
        compiler_params=pltpu.CompilerParams(needs_layout_passes=False),
    )


_sc_half = _make_sc(48, 24)


def _tc_graph(xs, weight):
    nslices, n, d = xs.shape
    return pl.pallas_call(
        _tc_graph_body,
        grid=(nslices,),
        in_specs=[
            pl.BlockSpec((1, n, d), lambda i: (i, 0, 0)),
            pl.BlockSpec((2, d, weight.shape[2]), lambda i: (0, 0, 0)),
        ],
        out_specs=pl.BlockSpec((NROWPAD, NPAD), lambda i: (i, 0)),
        out_shape=jax.ShapeDtypeStruct((nslices * NROWPAD, NPAD),
                                       jnp.float32),
    )(xs, weight)


def kernel(c_input, weight):
    b, t, n, d = c_input.shape
    xs = c_input.reshape(b * t, n, d)
    half = (b * t) // 2
    g1 = _tc_graph(xs[:half], weight)
    g2 = _tc_graph(xs[half:], weight)
    o1 = _sc_half(g1)
    o2 = _sc_half(g2)
    out = jnp.concatenate([o1, o2], axis=0)
    return out.reshape(b * t, NROWPAD, NPAD)[:, :n, :n].reshape(b, t, n, n)
